# baseline (speedup 1.0000x reference)
import functools

import numpy as np
import jax
import jax.numpy as jnp
from jax import lax
from jax.experimental import pallas as pl
from jax.experimental.pallas import tpu as pltpu
from jax.experimental.pallas import tpu_sc as plsc

_SIGMA = 0.2
_B = 64
_ROW = 56 * 56 * 192
_NC, _NS = 2, 16
_NW = _NC * _NS
_W = _ROW // _NW
_NV = _W // 16

_PERM = (17, 27, 42, 32, 1, 3, 58, 51, 40, 28, 52, 19, 9, 33, 11, 45,
         31, 5, 15, 39, 50, 47, 20, 0, 46, 14, 49, 44, 38, 61, 2, 54,
         36, 35, 62, 63, 21, 59, 30, 43, 22, 18, 24, 26, 53, 12, 16, 6,
         7, 57, 55, 48, 13, 37, 60, 10, 29, 34, 25, 56, 4, 41, 23, 8)


def _cycles(perm):
    seen, out = [False] * len(perm), []
    for s in range(len(perm)):
        if seen[s]:
            continue
        c, j = [s], perm[s]
        seen[s] = True
        while j != s:
            c.append(j)
            seen[j] = True
            j = perm[j]
        out.append(c)
    return out


_CYCLES = _cycles(_PERM)

_cache = {}


def _coeffs():
    key = jax.random.key(42)
    _, k_r, k_theta = jax.random.split(key, 3)
    rs = (1, 56, 56, 192)
    r = jax.random.normal(k_r, rs, dtype=jnp.float16) * jnp.float16(_SIGMA)
    theta = jax.random.uniform(k_theta, rs, dtype=jnp.float16,
                               minval=-np.pi, maxval=np.pi)
    a = (jnp.float16(1.0) + r * jnp.cos(theta)).astype(jnp.float32).reshape(_ROW)
    b = (r * jnp.sin(theta)).astype(jnp.float32).reshape(_ROW)
    return a, b


def _pack_coeffs(a, b):
    a16 = lax.bitcast_convert_type(a.astype(jnp.bfloat16), jnp.uint16)
    b16 = lax.bitcast_convert_type(b.astype(jnp.bfloat16), jnp.uint16)
    packed = (b16.astype(jnp.uint32) << 16) | a16.astype(jnp.uint32)
    return lax.bitcast_convert_type(packed, jnp.int32)


def _build():
    mesh = plsc.VectorSubcoreMesh(core_axis_name="c", subcore_axis_name="s")

    ring_rows, cyc_spans = [], []
    for cyc in _CYCLES:
        cyc_spans.append((len(ring_rows), len(cyc), cyc))
        ring_rows.extend(cyc)
        ring_rows.append(cyc[0])
    n_loads = len(ring_rows)

    @functools.partial(
        pl.kernel,
        mesh=mesh,
        out_type=jax.ShapeDtypeStruct((_B * _ROW,), jnp.float32),
        scratch_types=[
            pltpu.VMEM((_W,), jnp.int32),
            pltpu.VMEM((_W,), jnp.float32),
            pltpu.VMEM((_W,), jnp.float32),
            pltpu.VMEM((_W,), jnp.float32),
            pltpu.VMEM((_W,), jnp.float32),
            pltpu.SemaphoreType.DMA,
            pltpu.SemaphoreType.DMA,
            pltpu.SemaphoreType.DMA,
            pltpu.SemaphoreType.DMA,
        ],
    )
    def mixfeat(x_hbm, c_hbm, y_hbm, c_v, xr0, xr1, or0, or1, xs0, xs1, os0, os1):
        wid = lax.axis_index("s") * _NC + lax.axis_index("c")
        base = wid * _W
        xr, orr, xsem, osem = (xr0, xr1), (or0, or1), (xs0, xs1), (os0, os1)

        pltpu.sync_copy(c_hbm.at[pl.ds(base, _W)], c_v)

        def compute(xa_ref, xb_ref, o_ref):
            def body(v, _):
                s = pl.ds(v * 16, 16)
                cc = c_v[s]
                av = lax.bitcast_convert_type(lax.shift_left(cc, 16),
                                              jnp.float32)
                bv = lax.bitcast_convert_type(
                    lax.bitwise_and(cc, jnp.int32(-65536)), jnp.float32)
                o_ref[s] = xa_ref[s] * av + xb_ref[s] * bv
                return _
            lax.fori_loop(0, _NV, body, None, unroll=4)

        load_h = [None] * n_loads
        load_waited = [False] * n_loads

        def issue_load(li):
            if li < n_loads:
                load_h[li] = pltpu.async_copy(
                    x_hbm.at[pl.ds(ring_rows[li] * _ROW + base, _W)],
                    xr[li % 2], xsem[li % 2])

        def wait_load(li):
            if not load_waited[li]:
                load_h[li].wait()
                load_waited[li] = True

        issue_load(0)
        issue_load(1)
        store_h = [None, None]
        q = 0
        for start, m, cyc in cyc_spans:
            for k in range(m):
                a_li, b_li = start + k, start + k + 1
                wait_load(a_li)
                wait_load(b_li)
                if store_h[q % 2] is not None:
                    store_h[q % 2].wait()
                compute(xr[a_li % 2], xr[b_li % 2], orr[q % 2])
                store_h[q % 2] = pltpu.async_copy(
                    orr[q % 2],
                    y_hbm.at[pl.ds(cyc[k] * _ROW + base, _W)],
                    osem[q % 2])
                if k < m - 1:
                    issue_load(a_li + 2)
                else:
                    issue_load(start + m + 1)
                    issue_load(start + m + 2)
                q += 1
        store_h[0].wait()
        store_h[1].wait()

    return mixfeat


def kernel(inputs):
    if "f" not in _cache:
        _cache["f"] = _build()
    a, b = _coeffs()
    c = _pack_coeffs(a, b)
    x = inputs.reshape(_B * _ROW)
    y = _cache["f"](x, c)
    return y.reshape(inputs.shape)


# speedup vs baseline: 1.5342x; 1.5342x over previous
import functools

import numpy as np
import jax
import jax.numpy as jnp
from jax import lax
from jax.experimental import pallas as pl
from jax.experimental.pallas import tpu as pltpu
from jax.experimental.pallas import tpu_sc as plsc

_SIGMA = 0.2
_B = 64
_ROW = 56 * 56 * 192
_NC, _NS = 2, 16
_NW = _NC * _NS
_W = _ROW // _NW
_NV = _W // 16

_PERM = (17, 27, 42, 32, 1, 3, 58, 51, 40, 28, 52, 19, 9, 33, 11, 45,
         31, 5, 15, 39, 50, 47, 20, 0, 46, 14, 49, 44, 38, 61, 2, 54,
         36, 35, 62, 63, 21, 59, 30, 43, 22, 18, 24, 26, 53, 12, 16, 6,
         7, 57, 55, 48, 13, 37, 60, 10, 29, 34, 25, 56, 4, 41, 23, 8)


def _cycles(perm):
    seen, out = [False] * len(perm), []
    for s in range(len(perm)):
        if seen[s]:
            continue
        c, j = [s], perm[s]
        seen[s] = True
        while j != s:
            c.append(j)
            seen[j] = True
            j = perm[j]
        out.append(c)
    return out


_CYCLES = _cycles(_PERM)

_cache = {}


def _coeffs():
    key = jax.random.key(42)
    _, k_r, k_theta = jax.random.split(key, 3)
    rs = (1, 56, 56, 192)
    r = jax.random.normal(k_r, rs, dtype=jnp.float16) * jnp.float16(_SIGMA)
    theta = jax.random.uniform(k_theta, rs, dtype=jnp.float16,
                               minval=-np.pi, maxval=np.pi)
    a = (jnp.float16(1.0) + r * jnp.cos(theta)).astype(jnp.float32).reshape(_ROW)
    b = (r * jnp.sin(theta)).astype(jnp.float32).reshape(_ROW)
    return a, b


def _pack_coeffs(a, b):
    a16 = lax.bitcast_convert_type(a.astype(jnp.bfloat16), jnp.uint16)
    b16 = lax.bitcast_convert_type(b.astype(jnp.bfloat16), jnp.uint16)
    packed = (b16.astype(jnp.uint32) << 16) | a16.astype(jnp.uint32)
    return lax.bitcast_convert_type(packed, jnp.int32)


def _build():
    mesh = plsc.VectorSubcoreMesh(core_axis_name="c", subcore_axis_name="s")

    ring_rows, cyc_spans = [], []
    for cyc in _CYCLES:
        cyc_spans.append((len(ring_rows), len(cyc), cyc))
        ring_rows.extend(cyc)
        ring_rows.append(cyc[0])
    n_loads = len(ring_rows)

    @functools.partial(
        pl.kernel,
        mesh=mesh,
        out_type=jax.ShapeDtypeStruct((_B * _ROW,), jnp.float32),
        scratch_types=[
            pltpu.VMEM((_W,), jnp.int32),
            pltpu.VMEM((_W,), jnp.float32),
            pltpu.VMEM((_W,), jnp.float32),
            pltpu.VMEM((_W,), jnp.float32),
            pltpu.VMEM((_W,), jnp.float32),
            pltpu.SemaphoreType.DMA,
            pltpu.SemaphoreType.DMA,
            pltpu.SemaphoreType.DMA,
            pltpu.SemaphoreType.DMA,
        ],
    )
    def mixfeat(x_hbm, c_hbm, y_hbm, c_v, xr0, xr1, or0, or1, xs0, xs1, os0, os1):
        wid = lax.axis_index("s") * _NC + lax.axis_index("c")
        base = wid * _W
        xr, orr, xsem, osem = (xr0, xr1), (or0, or1), (xs0, xs1), (os0, os1)

        pltpu.sync_copy(c_hbm.at[pl.ds(base, _W)], c_v)

        def compute(xa_ref, xb_ref, o_ref):
            @plsc.parallel_loop(0, _W, 16, unroll=8)
            def body(v):
                s = pl.ds(v, 16)
                cc = c_v[s]
                av = lax.bitcast_convert_type(lax.shift_left(cc, 16),
                                              jnp.float32)
                bv = lax.bitcast_convert_type(
                    lax.bitwise_and(cc, jnp.int32(-65536)), jnp.float32)
                o_ref[s] = xa_ref[s] * av + xb_ref[s] * bv

        load_h = [None] * n_loads
        load_waited = [False] * n_loads

        def issue_load(li):
            if li < n_loads:
                load_h[li] = pltpu.async_copy(
                    x_hbm.at[pl.ds(ring_rows[li] * _ROW + base, _W)],
                    xr[li % 2], xsem[li % 2])

        def wait_load(li):
            if not load_waited[li]:
                load_h[li].wait()
                load_waited[li] = True

        issue_load(0)
        issue_load(1)
        store_h = [None, None]
        q = 0
        for start, m, cyc in cyc_spans:
            for k in range(m):
                a_li, b_li = start + k, start + k + 1
                wait_load(a_li)
                wait_load(b_li)
                if store_h[q % 2] is not None:
                    store_h[q % 2].wait()
                compute(xr[a_li % 2], xr[b_li % 2], orr[q % 2])
                store_h[q % 2] = pltpu.async_copy(
                    orr[q % 2],
                    y_hbm.at[pl.ds(cyc[k] * _ROW + base, _W)],
                    osem[q % 2])
                if k < m - 1:
                    issue_load(a_li + 2)
                else:
                    issue_load(start + m + 1)
                    issue_load(start + m + 2)
                q += 1
        store_h[0].wait()
        store_h[1].wait()

    return mixfeat


def kernel(inputs):
    if "f" not in _cache:
        _cache["f"] = _build()
    a, b = _coeffs()
    c = _pack_coeffs(a, b)
    x = inputs.reshape(_B * _ROW)
    y = _cache["f"](x, c)
    return y.reshape(inputs.shape)
